# baseline (speedup 1.0000x reference)
import functools

import jax
import jax.numpy as jnp
from jax import lax
from jax.experimental import pallas as pl
from jax.experimental.pallas import tpu as pltpu
from jax.experimental.pallas import tpu_sc as plsc

VOCAB = 1000
HEAD = 896
TAIL = 128
TAIL0 = 880
BATCH = 1024
SEQ = 200
NSPLIT = 4
BSLICE = BATCH // NSPLIT
N_ROWS = BSLICE * SEQ
NUM_WORKERS = 32
ROWS_PER_W = N_ROWS // NUM_WORKERS
CHUNK = 40
N_CHUNKS = ROWS_PER_W // CHUNK
N_PAIRS = N_CHUNKS // 2
CHUNKS_PER_B = SEQ // CHUNK
B_PER_W = ROWS_PER_W // SEQ


def _emb_body(x_hbm, head_hbm, tail_hbm, out_hbm, idx_v, stage_a, stage_b,
              tbuf_a, tbuf_b, gsem, wsem):
    wid = lax.axis_index("s") * 2 + lax.axis_index("c")
    pltpu.sync_copy(x_hbm.at[pl.ds(wid * ROWS_PER_W, ROWS_PER_W)], idx_v)

    def gather(j, buf, tbuf):
        idx = idx_v.at[pl.ds(j * CHUNK, CHUNK)]
        pltpu.async_copy(head_hbm.at[idx], buf.at[:, pl.ds(0, HEAD)], gsem)
        pltpu.async_copy(tail_hbm.at[idx], tbuf, gsem)

    def drain_gather(buf, tbuf):
        pltpu.make_async_copy(
            head_hbm.at[pl.ds(0, CHUNK)], buf.at[:, pl.ds(0, HEAD)], gsem).wait()
        pltpu.make_async_copy(tail_hbm.at[pl.ds(0, CHUNK)], tbuf, gsem).wait()

    def fix_tail(buf, tbuf):
        cols = 992 + lax.iota(jnp.int32, 16)
        msk = cols < VOCAB

        def row(r, carry):
            for k in range(6):
                buf[r, pl.ds(896 + 16 * k, 16)] = tbuf[r, pl.ds(16 + 16 * k, 16)]
            xv = tbuf[r, pl.ds(112, 16)]
            rows16 = jnp.full((16,), r, jnp.int32)
            plsc.store_scatter(buf, [rows16, cols], xv, mask=msk)
            return carry

        lax.fori_loop(0, CHUNK, row, 0)

    def write(j, buf):
        b = wid * B_PER_W + j // CHUNKS_PER_B
        s0 = (j % CHUNKS_PER_B) * CHUNK
        return pltpu.async_copy(buf, out_hbm.at[b, pl.ds(s0, CHUNK)], wsem)

    gather(0, stage_a, tbuf_a)

    def body(t, carry):
        j0 = 2 * t
        gather(j0 + 1, stage_b, tbuf_b)
        drain_gather(stage_a, tbuf_a)
        fix_tail(stage_a, tbuf_a)
        wa = write(j0, stage_a)
        drain_gather(stage_b, tbuf_b)
        fix_tail(stage_b, tbuf_b)
        wb = write(j0 + 1, stage_b)
        wa.wait()

        @pl.when(t + 1 < N_PAIRS)
        def _():
            gather(j0 + 2, stage_a, tbuf_a)

        wb.wait()
        return carry

    lax.fori_loop(0, N_PAIRS, body, 0)


@jax.jit
def _emb_call(x_flat, head, tail):
    mesh = plsc.VectorSubcoreMesh(core_axis_name="c", subcore_axis_name="s")
    f = functools.partial(
        pl.kernel,
        mesh=mesh,
        out_type=jax.ShapeDtypeStruct((BSLICE, SEQ, VOCAB), jnp.float32),
        scratch_types=[
            pltpu.VMEM((ROWS_PER_W,), jnp.int32),
            pltpu.VMEM((CHUNK, VOCAB), jnp.float32),
            pltpu.VMEM((CHUNK, VOCAB), jnp.float32),
            pltpu.VMEM((CHUNK, TAIL), jnp.float32),
            pltpu.VMEM((CHUNK, TAIL), jnp.float32),
            pltpu.SemaphoreType.DMA,
            pltpu.SemaphoreType.DMA,
        ],
        compiler_params=pltpu.CompilerParams(needs_layout_passes=False),
    )(_emb_body)
    return f(x_flat, head, tail)


def kernel(x, table):
    xs = x.reshape(NSPLIT, N_ROWS).astype(jnp.int32)
    head = table[:, :HEAD]
    tail = jnp.pad(table[:, TAIL0:], ((0, 0), (0, TAIL0 + TAIL - VOCAB)))
    out = jnp.zeros((BATCH, SEQ, VOCAB), jnp.float32)
    for i in range(NSPLIT):
        out = lax.dynamic_update_slice(
            out, _emb_call(xs[i], head, tail), (i * BSLICE, 0, 0))
    return out


# speedup vs baseline: 1.3950x; 1.3950x over previous
import functools

import jax
import jax.numpy as jnp
from jax import lax
from jax.experimental import pallas as pl
from jax.experimental.pallas import tpu as pltpu
from jax.experimental.pallas import tpu_sc as plsc

VOCAB = 1000
HEAD = 896
TAIL = 128
TAIL0 = 880
BATCH = 1024
SEQ = 200
NSPLIT = 1
BSLICE = BATCH // NSPLIT
N_ROWS = BSLICE * SEQ
NUM_WORKERS = 32
ROWS_PER_W = N_ROWS // NUM_WORKERS
CHUNK = 40
N_CHUNKS = ROWS_PER_W // CHUNK
N_PAIRS = N_CHUNKS // 2
CHUNKS_PER_B = SEQ // CHUNK
B_PER_W = ROWS_PER_W // SEQ


def _emb_body(x_hbm, head_hbm, tail_hbm, out_hbm, idx_v, stage_a, stage_b,
              tbuf_a, tbuf_b, gsem, wsem):
    wid = lax.axis_index("s") * 2 + lax.axis_index("c")
    pltpu.sync_copy(x_hbm.at[pl.ds(wid * ROWS_PER_W, ROWS_PER_W)], idx_v)

    def gather(j, buf, tbuf):
        idx = idx_v.at[pl.ds(j * CHUNK, CHUNK)]
        pltpu.async_copy(head_hbm.at[idx], buf.at[:, pl.ds(0, HEAD)], gsem)
        pltpu.async_copy(tail_hbm.at[idx], tbuf, gsem)

    def drain_gather(buf, tbuf):
        pltpu.make_async_copy(
            head_hbm.at[pl.ds(0, CHUNK)], buf.at[:, pl.ds(0, HEAD)], gsem).wait()
        pltpu.make_async_copy(tail_hbm.at[pl.ds(0, CHUNK)], tbuf, gsem).wait()

    def fix_tail(buf, tbuf):
        cols = 992 + lax.iota(jnp.int32, 16)
        msk = cols < VOCAB

        def row(r, carry):
            for k in range(6):
                buf[r, pl.ds(896 + 16 * k, 16)] = tbuf[r, pl.ds(16 + 16 * k, 16)]
            xv = tbuf[r, pl.ds(112, 16)]
            rows16 = jnp.full((16,), r, jnp.int32)
            plsc.store_scatter(buf, [rows16, cols], xv, mask=msk)
            return carry

        lax.fori_loop(0, CHUNK, row, 0)

    def write(j, buf):
        b = wid * B_PER_W + j // CHUNKS_PER_B
        s0 = (j % CHUNKS_PER_B) * CHUNK
        return pltpu.async_copy(buf, out_hbm.at[b, pl.ds(s0, CHUNK)], wsem)

    gather(0, stage_a, tbuf_a)

    def body(t, carry):
        j0 = 2 * t
        gather(j0 + 1, stage_b, tbuf_b)
        drain_gather(stage_a, tbuf_a)
        fix_tail(stage_a, tbuf_a)
        wa = write(j0, stage_a)
        drain_gather(stage_b, tbuf_b)
        fix_tail(stage_b, tbuf_b)
        wb = write(j0 + 1, stage_b)
        wa.wait()

        @pl.when(t + 1 < N_PAIRS)
        def _():
            gather(j0 + 2, stage_a, tbuf_a)

        wb.wait()
        return carry

    lax.fori_loop(0, N_PAIRS, body, 0)


@jax.jit
def _emb_call(x_flat, head, tail):
    mesh = plsc.VectorSubcoreMesh(core_axis_name="c", subcore_axis_name="s")
    f = functools.partial(
        pl.kernel,
        mesh=mesh,
        out_type=jax.ShapeDtypeStruct((BSLICE, SEQ, VOCAB), jnp.float32),
        scratch_types=[
            pltpu.VMEM((ROWS_PER_W,), jnp.int32),
            pltpu.VMEM((CHUNK, VOCAB), jnp.float32),
            pltpu.VMEM((CHUNK, VOCAB), jnp.float32),
            pltpu.VMEM((CHUNK, TAIL), jnp.float32),
            pltpu.VMEM((CHUNK, TAIL), jnp.float32),
            pltpu.SemaphoreType.DMA,
            pltpu.SemaphoreType.DMA,
        ],
        compiler_params=pltpu.CompilerParams(needs_layout_passes=False),
    )(_emb_body)
    return f(x_flat, head, tail)


def kernel(x, table):
    x_flat = x.reshape(N_ROWS).astype(jnp.int32)
    head = table[:, :HEAD]
    tail = jnp.pad(table[:, TAIL0:], ((0, 0), (0, TAIL0 + TAIL - VOCAB)))
    return _emb_call(x_flat, head, tail)


# speedup vs baseline: 1.4000x; 1.0036x over previous
import functools

import jax
import jax.numpy as jnp
from jax import lax
from jax.experimental import pallas as pl
from jax.experimental.pallas import tpu as pltpu
from jax.experimental.pallas import tpu_sc as plsc

VOCAB = 1000
HEAD = 896
TAIL = 128
TAIL0 = 880
BATCH = 1024
SEQ = 200
NSPLIT = 1
BSLICE = BATCH // NSPLIT
N_ROWS = BSLICE * SEQ
NUM_WORKERS = 32
ROWS_PER_W = N_ROWS // NUM_WORKERS
CHUNK = 40
N_CHUNKS = ROWS_PER_W // CHUNK
N_PAIRS = N_CHUNKS // 2
CHUNKS_PER_B = SEQ // CHUNK
B_PER_W = ROWS_PER_W // SEQ


def _emb_body(x_hbm, head_hbm, tail_hbm, out_hbm, idx_v, stage_a, stage_b,
              tbuf_a, tbuf_b, gsem, wsem):
    wid = lax.axis_index("s") * 2 + lax.axis_index("c")
    pltpu.sync_copy(x_hbm.at[pl.ds(wid * ROWS_PER_W, ROWS_PER_W)], idx_v)

    def gather(j, buf, tbuf):
        idx = idx_v.at[pl.ds(j * CHUNK, CHUNK)]
        pltpu.async_copy(head_hbm.at[idx], buf.at[:, pl.ds(0, HEAD)], gsem)
        pltpu.async_copy(tail_hbm.at[idx], tbuf, gsem)

    def drain_gather(buf, tbuf):
        pltpu.make_async_copy(
            head_hbm.at[pl.ds(0, CHUNK)], buf.at[:, pl.ds(0, HEAD)], gsem).wait()
        pltpu.make_async_copy(tail_hbm.at[pl.ds(0, CHUNK)], tbuf, gsem).wait()

    def fix_tail(buf, tbuf):
        cols = 992 + lax.iota(jnp.int32, 16)
        msk = cols < VOCAB
        lanes = lax.iota(jnp.int32, 16)

        for r in range(CHUNK):
            for k in range(6):
                buf[r, pl.ds(896 + 16 * k, 16)] = tbuf[r, pl.ds(16 + 16 * k, 16)]
        for g in range(0, CHUNK, 16):
            rows16 = g + lanes
            mg = rows16 < CHUNK
            for c in range(8):
                xv = plsc.load_gather(
                    tbuf, [rows16, jnp.full((16,), 112 + c, jnp.int32)], mask=mg)
                plsc.store_scatter(
                    buf, [rows16, jnp.full((16,), 992 + c, jnp.int32)], xv, mask=mg)

    def write(j, buf):
        b = wid * B_PER_W + j // CHUNKS_PER_B
        s0 = (j % CHUNKS_PER_B) * CHUNK
        return pltpu.async_copy(buf, out_hbm.at[b, pl.ds(s0, CHUNK)], wsem)

    gather(0, stage_a, tbuf_a)

    def body(t, carry):
        j0 = 2 * t
        gather(j0 + 1, stage_b, tbuf_b)
        drain_gather(stage_a, tbuf_a)
        fix_tail(stage_a, tbuf_a)
        wa = write(j0, stage_a)
        drain_gather(stage_b, tbuf_b)
        fix_tail(stage_b, tbuf_b)
        wb = write(j0 + 1, stage_b)
        wa.wait()

        @pl.when(t + 1 < N_PAIRS)
        def _():
            gather(j0 + 2, stage_a, tbuf_a)

        wb.wait()
        return carry

    lax.fori_loop(0, N_PAIRS, body, 0)


@jax.jit
def _emb_call(x_flat, head, tail):
    mesh = plsc.VectorSubcoreMesh(core_axis_name="c", subcore_axis_name="s")
    f = functools.partial(
        pl.kernel,
        mesh=mesh,
        out_type=jax.ShapeDtypeStruct((BSLICE, SEQ, VOCAB), jnp.float32),
        scratch_types=[
            pltpu.VMEM((ROWS_PER_W,), jnp.int32),
            pltpu.VMEM((CHUNK, VOCAB), jnp.float32),
            pltpu.VMEM((CHUNK, VOCAB), jnp.float32),
            pltpu.VMEM((CHUNK, TAIL), jnp.float32),
            pltpu.VMEM((CHUNK, TAIL), jnp.float32),
            pltpu.SemaphoreType.DMA,
            pltpu.SemaphoreType.DMA,
        ],
        compiler_params=pltpu.CompilerParams(needs_layout_passes=False),
    )(_emb_body)
    return f(x_flat, head, tail)


def kernel(x, table):
    x_flat = x.reshape(N_ROWS).astype(jnp.int32)
    head = table[:, :HEAD]
    tail = jnp.pad(table[:, TAIL0:], ((0, 0), (0, TAIL0 + TAIL - VOCAB)))
    return _emb_call(x_flat, head, tail)
